# Initial kernel scaffold; baseline (speedup 1.0000x reference)
import jax
import jax.numpy as jnp
from jax.experimental import pallas as pl


def kernel(x, x_struct, x_e, edge_index, W1, b1, W2, b2, Wc1, bc1, Wc2, bc2):
    raise NotImplementedError("write your pallas kernel here")


# speedup vs baseline: 1.5398x; 1.5398x over previous
import functools

import jax
import jax.numpy as jnp
from jax.experimental import pallas as pl
from jax.experimental.pallas import tpu as pltpu

N_BLK = 400


def _mm1_body(xs_ref, w1_ref, b1_ref, h_ref):
    h_ref[...] = jnp.tanh(
        jnp.dot(xs_ref[...], w1_ref[...], preferred_element_type=jnp.float32)
        + b1_ref[...]
    )


def _mm2_body(h_ref, s_ref, cnt_ref, w2_ref, b2_ref, h2_ref):
    agg = s_ref[...] / jnp.maximum(cnt_ref[...], 1.0)
    h2_ref[...] = jnp.tanh(
        jnp.dot(h_ref[...] + agg, w2_ref[...], preferred_element_type=jnp.float32)
        + b2_ref[...]
    )


def _mm3_body(m_ref, wc1_ref, bc1_ref, wc2_ref, bc2_ref, out_ref):
    z = jax.nn.relu(
        jnp.dot(m_ref[...], wc1_ref[...], preferred_element_type=jnp.float32)
        + bc1_ref[...]
    )
    out_ref[...] = (
        jnp.dot(z, wc2_ref[...], preferred_element_type=jnp.float32) + bc2_ref[...]
    )


def _row_spec(d):
    return pl.BlockSpec((N_BLK, d), lambda i: (i, 0))


def _full_spec(a, b):
    return pl.BlockSpec((a, b), lambda i: (0, 0))


def kernel(x, x_struct, x_e, edge_index, W1, b1, W2, b2, Wc1, bc1, Wc2, bc2):
    n = x_struct.shape[0]
    hid = W1.shape[1]
    out_d = Wc2.shape[1]
    grid = (n // N_BLK,)
    src = edge_index[0]
    dst = edge_index[1]

    h = pl.pallas_call(
        _mm1_body,
        grid=grid,
        in_specs=[
            _row_spec(x_struct.shape[1]),
            _full_spec(x_struct.shape[1], hid),
            pl.BlockSpec((1, hid), lambda i: (0, 0)),
        ],
        out_specs=_row_spec(hid),
        out_shape=jax.ShapeDtypeStruct((n, hid), jnp.float32),
    )(x_struct, W1, b1.reshape(1, hid))

    s = jax.ops.segment_sum(h[src], dst, num_segments=n)
    cnt = jax.ops.segment_sum(jnp.ones((src.shape[0], 1), jnp.float32), dst,
                              num_segments=n)

    h2 = pl.pallas_call(
        _mm2_body,
        grid=grid,
        in_specs=[
            _row_spec(hid),
            _row_spec(hid),
            pl.BlockSpec((N_BLK, 1), lambda i: (i, 0)),
            _full_spec(hid, hid),
            pl.BlockSpec((1, hid), lambda i: (0, 0)),
        ],
        out_specs=_row_spec(hid),
        out_shape=jax.ShapeDtypeStruct((n, hid), jnp.float32),
    )(h, s, cnt, W2, b2.reshape(1, hid))

    mn = jax.ops.segment_min(h2[src], dst, num_segments=n)
    m = jnp.where(cnt > 0, mn, 0.0)

    return pl.pallas_call(
        _mm3_body,
        grid=grid,
        in_specs=[
            _row_spec(hid),
            _full_spec(hid, hid),
            pl.BlockSpec((1, hid), lambda i: (0, 0)),
            _full_spec(hid, out_d),
            pl.BlockSpec((1, out_d), lambda i: (0, 0)),
        ],
        out_specs=_row_spec(out_d),
        out_shape=jax.ShapeDtypeStruct((n, out_d), jnp.float32),
    )(m, Wc1, bc1.reshape(1, hid), Wc2, bc2.reshape(1, out_d))


# speedup vs baseline: 2.8241x; 1.8340x over previous
import functools

import jax
import jax.numpy as jnp
from jax import lax
from jax.experimental import pallas as pl
from jax.experimental.pallas import tpu as pltpu
from jax.experimental.pallas import tpu_sc as plsc

N_BLK = 400
NC = 2
NS = 16
K_E = 128
ACC_ROWS = 50048
TRASH_ROW = 50040


def _mm1_body(xs_ref, w1_ref, b1_ref, ha_ref, hb_ref):
    h = jnp.tanh(
        jnp.dot(xs_ref[...], w1_ref[...], preferred_element_type=jnp.float32)
        + b1_ref[...]
    )
    ha_ref[...] = h[:, :32]
    hb_ref[...] = h[:, 32:]


def _mm2_body(ha_ref, hb_ref, sa_ref, sb_ref, cnt_ref, w2_ref, b2_ref, h2_ref):
    h = jnp.concatenate([ha_ref[...], hb_ref[...]], axis=1)
    s = jnp.concatenate([sa_ref[...], sb_ref[...]], axis=1)
    agg = s / jnp.maximum(cnt_ref[...][:, 0:1], 1.0)
    h2_ref[...] = jnp.tanh(
        jnp.dot(h + agg, w2_ref[...], preferred_element_type=jnp.float32)
        + b2_ref[...]
    )


def _mm3_body(m_ref, wc1_ref, bc1_ref, wc2_ref, bc2_ref, out_ref):
    z = jax.nn.relu(
        jnp.dot(m_ref[...], wc1_ref[...], preferred_element_type=jnp.float32)
        + bc1_ref[...]
    )
    out_ref[...] = (
        jnp.dot(z, wc2_ref[...], preferred_element_type=jnp.float32) + bc2_ref[...]
    )


Z_ROWS = 136


def _zero_init(zt, acc, s):
    rows_per_sub = ACC_ROWS // NS

    def z(j, carry):
        pltpu.sync_copy(zt, acc.at[pl.ds(s * rows_per_sub + j * Z_ROWS, Z_ROWS)])
        return carry

    lax.fori_loop(0, rows_per_sub // Z_ROWS, z, 0)


def _seg_sum_body(ha, hb, src_h, dst_h, zt,
                  sa, sb,
                  srcv, dstv, rows, acc, sem):
    c = lax.axis_index("c")
    s = lax.axis_index("s")
    is0 = c == 0
    n_e = src_h.shape[0]
    per_sub = n_e // (NS * K_E)

    sl = pl.ds(s * (ACC_ROWS // NS), ACC_ROWS // NS)
    _zero_init(zt, acc, s)
    plsc.subcore_barrier()

    def chunk(k, carry):
        base = (s + NS * k) * K_E
        pltpu.sync_copy(src_h.at[pl.ds(base, K_E)], srcv)
        pltpu.sync_copy(dst_h.at[pl.ds(base, K_E)], dstv)

        @pl.when(is0)
        def _():
            pltpu.async_copy(ha.at[srcv], rows, sem).wait()

        @pl.when(jnp.logical_not(is0))
        def _():
            pltpu.async_copy(hb.at[srcv], rows, sem).wait()

        pltpu.sync_copy(rows, acc.at[dstv], add=True)
        return carry

    lax.fori_loop(0, per_sub, chunk, 0)
    plsc.subcore_barrier()

    @pl.when(is0)
    def _():
        pltpu.sync_copy(acc.at[sl], sa.at[sl])

    @pl.when(jnp.logical_not(is0))
    def _():
        pltpu.sync_copy(acc.at[sl], sb.at[sl])


def _count_body(dst_h, ztc, ones_h, cnt, dstv, onesv, cntsh):
    c = lax.axis_index("c")
    s = lax.axis_index("s")
    is0 = c == 0
    n_e = dst_h.shape[0]
    rows_per_sub = ACC_ROWS // NS

    pltpu.sync_copy(ones_h, onesv)

    @pl.when(is0)
    def _():
        def z(j, carry):
            pltpu.sync_copy(
                ztc, cntsh.at[pl.ds(s * rows_per_sub + j * Z_ROWS, Z_ROWS)]
            )
            return carry

        lax.fori_loop(0, rows_per_sub // Z_ROWS, z, 0)

    plsc.subcore_barrier()

    @pl.when(is0)
    def _():
        def chunk(k, carry):
            base = (s + NS * k) * K_E
            pltpu.sync_copy(dst_h.at[pl.ds(base, K_E)], dstv)
            pltpu.sync_copy(onesv, cntsh.at[dstv], add=True)
            return carry

        lax.fori_loop(0, n_e // (NS * K_E), chunk, 0)

    plsc.subcore_barrier()

    @pl.when(is0)
    def _():
        sl = pl.ds(s * rows_per_sub, rows_per_sub)
        pltpu.sync_copy(cntsh.at[sl], cnt.at[sl])


def _row_spec(d):
    return pl.BlockSpec((N_BLK, d), lambda i: (i, 0))


def _full_spec(a, b):
    return pl.BlockSpec((a, b), lambda i: (0, 0))


def kernel(x, x_struct, x_e, edge_index, W1, b1, W2, b2, Wc1, bc1, Wc2, bc2):
    n = x_struct.shape[0]
    hid = W1.shape[1]
    out_d = Wc2.shape[1]
    grid = (n // N_BLK,)

    ha, hb = pl.pallas_call(
        _mm1_body,
        grid=grid,
        in_specs=[
            _row_spec(x_struct.shape[1]),
            _full_spec(x_struct.shape[1], hid),
            pl.BlockSpec((1, hid), lambda i: (0, 0)),
        ],
        out_specs=[_row_spec(32), _row_spec(32)],
        out_shape=[
            jax.ShapeDtypeStruct((n, 32), jnp.float32),
            jax.ShapeDtypeStruct((n, 32), jnp.float32),
        ],
    )(x_struct, W1, b1.reshape(1, hid))

    mesh = plsc.VectorSubcoreMesh(
        core_axis_name="c", subcore_axis_name="s", num_cores=NC, num_subcores=NS
    )
    seg_sum = pl.kernel(
        _seg_sum_body,
        out_type=[
            jax.ShapeDtypeStruct((ACC_ROWS, 32), jnp.float32),
            jax.ShapeDtypeStruct((ACC_ROWS, 32), jnp.float32),
        ],
        mesh=mesh,
        scratch_types=[
            pltpu.VMEM((K_E,), jnp.int32),
            pltpu.VMEM((K_E,), jnp.int32),
            pltpu.VMEM((K_E, 32), jnp.float32),
            pltpu.VMEM_SHARED((ACC_ROWS, 32), jnp.float32),
            pltpu.SemaphoreType.DMA,
        ],
        compiler_params=pltpu.CompilerParams(use_tc_tiling_on_sc=False),
    )
    seg_cnt = pl.kernel(
        _count_body,
        out_type=jax.ShapeDtypeStruct((ACC_ROWS, 8), jnp.float32),
        mesh=mesh,
        scratch_types=[
            pltpu.VMEM((K_E,), jnp.int32),
            pltpu.VMEM((K_E, 8), jnp.float32),
            pltpu.VMEM_SHARED((ACC_ROWS, 8), jnp.float32),
        ],
        compiler_params=pltpu.CompilerParams(use_tc_tiling_on_sc=False),
    )
    zeros_t = jnp.zeros((Z_ROWS, 32), jnp.float32)
    zeros_tc = jnp.zeros((Z_ROWS, 8), jnp.float32)
    ones_h = jnp.ones((K_E, 8), jnp.float32)
    src = edge_index[0]
    dst = edge_index[1]
    e = src.shape[0]
    blk = K_E * NS
    e_pad = ((e + blk - 1) // blk) * blk
    src1 = jnp.concatenate([src, jnp.zeros((e_pad - e,), jnp.int32)])
    dst1 = jnp.concatenate([dst, jnp.full((e_pad - e,), TRASH_ROW, jnp.int32)])
    sa, sb = seg_sum(ha, hb, src1, dst1, zeros_t)
    cnt = seg_cnt(dst1, zeros_tc, ones_h)

    h2 = pl.pallas_call(
        _mm2_body,
        grid=grid,
        in_specs=[
            _row_spec(32),
            _row_spec(32),
            _row_spec(32),
            _row_spec(32),
            pl.BlockSpec((N_BLK, 8), lambda i: (i, 0)),
            _full_spec(hid, hid),
            pl.BlockSpec((1, hid), lambda i: (0, 0)),
        ],
        out_specs=_row_spec(hid),
        out_shape=jax.ShapeDtypeStruct((n, hid), jnp.float32),
    )(ha, hb, sa, sb, cnt, W2, b2.reshape(1, hid))

    mn = jax.ops.segment_min(h2[src], dst, num_segments=n)
    m = jnp.where(cnt[:n, 0:1] > 0, mn, 0.0)

    return pl.pallas_call(
        _mm3_body,
        grid=grid,
        in_specs=[
            _row_spec(hid),
            _full_spec(hid, hid),
            pl.BlockSpec((1, hid), lambda i: (0, 0)),
            _full_spec(hid, out_d),
            pl.BlockSpec((1, out_d), lambda i: (0, 0)),
        ],
        out_specs=_row_spec(out_d),
        out_shape=jax.ShapeDtypeStruct((n, out_d), jnp.float32),
    )(m, Wc1, bc1.reshape(1, hid), Wc2, bc2.reshape(1, out_d))
